# baseline (speedup 1.0000x reference)
import functools

import jax
import jax.numpy as jnp
from jax import lax
from jax.experimental import pallas as pl
from jax.experimental.pallas import tpu as pltpu
from jax.experimental.pallas import tpu_sc as plsc

N = 8192
D = 2048
H = 2048
E = 8
K = 2
C = 1000
CP = 1024

GATE_TB = 256
FEAT_TB = 512

NC = 2
NS = 16
L = 16
NW = NC * NS
TPW = N // NW

GB = 512
NBLK = N * K // GB + E
GROWS = NBLK * GB
BEXP_PAD = 80


def _gate_body(x_ref, wg_ref, bg_ref,
               gw_ref, idx_ref, e0_ref, e1_ref, r0_ref, r1_ref,
               w0_ref, w1_ref, cnt_ref, bexp_ref, cnt_scr):
    i = pl.program_id(0)

    @pl.when(i == 0)
    def _():
        cnt_scr[...] = jnp.zeros_like(cnt_scr)

    x = x_ref[...]
    logits = jnp.dot(x, wg_ref[...], preferred_element_type=jnp.float32)
    logits = logits + bg_ref[...]
    m = jnp.max(logits, axis=1, keepdims=True)
    ex = jnp.exp(logits - m)
    probs = ex / jnp.sum(ex, axis=1, keepdims=True)
    iota_e = lax.broadcasted_iota(jnp.int32, probs.shape, 1)
    m1 = jnp.max(probs, axis=1, keepdims=True)
    a1 = jnp.min(jnp.where(probs == m1, iota_e, E), axis=1, keepdims=True)
    masked = jnp.where(iota_e == a1, -jnp.inf, probs)
    m2 = jnp.max(masked, axis=1, keepdims=True)
    a2 = jnp.min(jnp.where(masked == m2, iota_e, E), axis=1, keepdims=True)
    gw = jnp.where(iota_e == a1, m1, 0.0) + jnp.where(iota_e == a2, m2, 0.0)
    gw_ref[...] = gw
    idx_ref[...] = jnp.concatenate([a1, a2], axis=1)
    e0_ref[...] = a1
    e1_ref[...] = a2
    w0_ref[...] = m1
    w1_ref[...] = m2

    sel = (jnp.where(iota_e == a1, 1.0, 0.0)
           + jnp.where(iota_e == a2, 1.0, 0.0))
    row_i = lax.broadcasted_iota(jnp.int32, (GATE_TB, GATE_TB), 0)
    col_i = lax.broadcasted_iota(jnp.int32, (GATE_TB, GATE_TB), 1)
    lt = jnp.where(col_i < row_i, 1.0, 0.0)
    rank_prev = jnp.dot(lt, sel, preferred_element_type=jnp.float32)
    runcnt = cnt_scr[...]
    rpt = rank_prev.astype(jnp.int32) + runcnt
    r0_ref[...] = jnp.sum(jnp.where(iota_e == a1, rpt, 0),
                          axis=1, keepdims=True)
    r1_ref[...] = jnp.sum(jnp.where(iota_e == a2, rpt, 0),
                          axis=1, keepdims=True)
    newcnt = runcnt + jnp.sum(sel, axis=0, keepdims=True).astype(jnp.int32)
    cnt_scr[...] = newcnt
    cnt_ref[...] = jnp.concatenate(
        [newcnt, jnp.zeros((1, 16 - E), jnp.int32)], axis=1)

    cntf = newcnt.astype(jnp.float32)
    pb = jnp.floor((cntf + (GB - 1)) * (1.0 / GB))
    r8 = lax.broadcasted_iota(jnp.int32, (E, E), 0)
    c8 = lax.broadcasted_iota(jnp.int32, (E, E), 1)
    slt = jnp.where(r8 < c8, 1.0, 0.0)
    excl_b = jnp.dot(pb, slt, preferred_element_type=jnp.float32)
    iv = lax.broadcasted_iota(jnp.int32, (BEXP_PAD, E), 0).astype(jnp.float32)
    hits = jnp.where(iv >= excl_b, 1, 0)
    bexp_ref[...] = jnp.sum(hits, axis=1, keepdims=True).astype(jnp.int32) - 1


def _backbone_body(x_ref, wb_ref, bb_ref, feat_ref):
    acc = jnp.dot(x_ref[...], wb_ref[...], preferred_element_type=jnp.float32)
    feat_ref[...] = jnp.maximum(acc + bb_ref[...], 0.0)


def _dispatch_body(feat_hbm, e0_hbm, e1_hbm, r0_hbm, r1_hbm, cnt_hbm,
                   g_hbm, pos0_hbm, pos1_hbm,
                   cntv, exclv, e0v, e1v, r0v, r1v, p0v, p1v,
                   fbuf0, fbuf1, lsem0, lsem1, ssem0, ssem1):
    wid = lax.axis_index("s") * NC + lax.axis_index("c")
    base = wid * TPW
    nchunk = TPW // L
    bufs = (fbuf0, fbuf1)
    lsems = (lsem0, lsem1)
    ssems = (ssem0, ssem1)

    pltpu.sync_copy(cnt_hbm, cntv)
    cnt = cntv[...]
    padded = ((cnt + (GB - 1)) // GB) * GB
    incl = plsc.cumsum(padded)
    exclv[...] = incl - padded

    pltpu.sync_copy(e0_hbm.at[pl.ds(base, TPW)], e0v)
    pltpu.sync_copy(e1_hbm.at[pl.ds(base, TPW)], e1v)
    pltpu.sync_copy(r0_hbm.at[pl.ds(base, TPW)], r0v)
    pltpu.sync_copy(r1_hbm.at[pl.ds(base, TPW)], r1v)

    for t in range(nchunk):
        sl = pl.ds(t * L, L)
        p0v[sl] = plsc.load_gather(exclv, [e0v[sl]]) + r0v[sl]
        p1v[sl] = plsc.load_gather(exclv, [e1v[sl]]) + r1v[sl]

    pltpu.sync_copy(p0v, pos0_hbm.at[pl.ds(base, TPW)])
    pltpu.sync_copy(p1v, pos1_hbm.at[pl.ds(base, TPW)])

    def _load(t):
        pltpu.make_async_copy(
            feat_hbm.at[pl.ds(base + t * L, L), :],
            bufs[t % 2], lsems[t % 2]).start()

    def _wait_load(t):
        pltpu.make_async_copy(
            feat_hbm.at[pl.ds(base + t * L, L), :],
            bufs[t % 2], lsems[t % 2]).wait()

    def _wait_scatter(t):
        sl = pl.ds(t * L, L)
        pltpu.make_async_copy(bufs[t % 2], g_hbm.at[p0v[sl]],
                              ssems[t % 2]).wait()
        pltpu.make_async_copy(bufs[t % 2], g_hbm.at[p1v[sl]],
                              ssems[t % 2]).wait()

    _load(0)
    for t in range(nchunk):
        sl = pl.ds(t * L, L)
        _wait_load(t)
        pltpu.make_async_copy(bufs[t % 2], g_hbm.at[p0v[sl]],
                              ssems[t % 2]).start()
        pltpu.make_async_copy(bufs[t % 2], g_hbm.at[p1v[sl]],
                              ssems[t % 2]).start()
        if t + 1 < nchunk:
            if t >= 1:
                _wait_scatter(t - 1)
            _load(t + 1)
    _wait_scatter(nchunk - 2)
    _wait_scatter(nchunk - 1)


def _expert_body(bexp_ref, g_ref, w_ref, b_ref, y_ref):
    i = pl.program_id(0)
    e = bexp_ref[i]
    gb = g_ref[...].astype(jnp.bfloat16)
    w = w_ref[pl.ds(e, 1)][0]
    b = b_ref[pl.ds(e, 1)][0]
    y = jnp.dot(gb, w, preferred_element_type=jnp.float32)
    y_ref[...] = jnp.pad(y + b, ((0, 0), (0, CP - C)))


def _slice_body(x_ref, o_ref):
    o_ref[...] = x_ref[:, :C]


def _combine_body(y_hbm, pos0_hbm, pos1_hbm, w0_hbm, w1_hbm, out_hbm,
                  p0v, p1v, w0v, w1v, ybuf0, ybuf1, obuf, sem0, sem1):
    wid = lax.axis_index("s") * NC + lax.axis_index("c")
    base = wid * TPW

    pltpu.sync_copy(pos0_hbm.at[pl.ds(base, TPW)], p0v)
    pltpu.sync_copy(pos1_hbm.at[pl.ds(base, TPW)], p1v)
    pltpu.sync_copy(w0_hbm.at[pl.ds(base, TPW)], w0v)
    pltpu.sync_copy(w1_hbm.at[pl.ds(base, TPW)], w1v)

    def chunk_body(cc, carry):
        sl = pl.ds(cc * L, L)
        cp0 = pltpu.async_copy(y_hbm.at[p0v[sl]], ybuf0, sem0)
        cp1 = pltpu.async_copy(y_hbm.at[p1v[sl]], ybuf1, sem1)
        cp0.wait()
        cp1.wait()

        def tok_body(tt, carry2):
            g = cc * L + tt
            w0s = plsc.load_gather(w0v, [jnp.full((L,), g, jnp.int32)])
            w1s = plsc.load_gather(w1v, [jnp.full((L,), g, jnp.int32)])
            for j in range(CP // L):
                js = pl.ds(j * L, L)
                obuf[tt, js] = w0s * ybuf0[tt, js] + w1s * ybuf1[tt, js]
            return carry2

        lax.fori_loop(0, L, tok_body, 0)
        pltpu.sync_copy(obuf, out_hbm.at[pl.ds(base + cc * L, L), :])
        return carry

    lax.fori_loop(0, TPW // L, chunk_body, 0)


@functools.lru_cache(maxsize=1)
def _sc_kernels():
    mesh = plsc.VectorSubcoreMesh(
        core_axis_name="c", subcore_axis_name="s",
        num_cores=NC, num_subcores=NS)

    dispatch = functools.partial(
        pl.kernel,
        out_type=[
            jax.ShapeDtypeStruct((GROWS, H), jnp.float32),
            jax.ShapeDtypeStruct((N,), jnp.int32),
            jax.ShapeDtypeStruct((N,), jnp.int32),
        ],
        mesh=mesh,
        compiler_params=pltpu.CompilerParams(needs_layout_passes=False),
        scratch_types=[
            pltpu.VMEM((16,), jnp.int32),
            pltpu.VMEM((16,), jnp.int32),
            pltpu.VMEM((TPW,), jnp.int32),
            pltpu.VMEM((TPW,), jnp.int32),
            pltpu.VMEM((TPW,), jnp.int32),
            pltpu.VMEM((TPW,), jnp.int32),
            pltpu.VMEM((TPW,), jnp.int32),
            pltpu.VMEM((TPW,), jnp.int32),
            pltpu.VMEM((L, H), jnp.float32),
            pltpu.VMEM((L, H), jnp.float32),
            pltpu.SemaphoreType.DMA,
            pltpu.SemaphoreType.DMA,
            pltpu.SemaphoreType.DMA,
            pltpu.SemaphoreType.DMA,
        ],
    )(_dispatch_body)

    combine = functools.partial(
        pl.kernel,
        out_type=jax.ShapeDtypeStruct((N, CP), jnp.float32),
        mesh=mesh,
        compiler_params=pltpu.CompilerParams(needs_layout_passes=False),
        scratch_types=[
            pltpu.VMEM((TPW,), jnp.int32),
            pltpu.VMEM((TPW,), jnp.int32),
            pltpu.VMEM((TPW,), jnp.float32),
            pltpu.VMEM((TPW,), jnp.float32),
            pltpu.VMEM((L, CP), jnp.float32),
            pltpu.VMEM((L, CP), jnp.float32),
            pltpu.VMEM((L, CP), jnp.float32),
            pltpu.SemaphoreType.DMA,
            pltpu.SemaphoreType.DMA,
        ],
    )(_combine_body)
    return dispatch, combine


def kernel(x, W_backbone, b_backbone, W_gate, b_gate, W_experts, b_experts):
    bg2 = b_gate.reshape(1, E)
    bb2 = b_backbone.reshape(1, H)

    (gate_weights, top_k_indices, e0, e1, r0, r1, w0, w1, cnt16,
     bexp2d) = pl.pallas_call(
        _gate_body,
        grid=(N // GATE_TB,),
        in_specs=[
            pl.BlockSpec((GATE_TB, D), lambda i: (i, 0)),
            pl.BlockSpec((D, E), lambda i: (0, 0)),
            pl.BlockSpec((1, E), lambda i: (0, 0)),
        ],
        out_specs=[
            pl.BlockSpec((GATE_TB, E), lambda i: (i, 0)),
            pl.BlockSpec((GATE_TB, K), lambda i: (i, 0)),
            pl.BlockSpec((GATE_TB, 1), lambda i: (i, 0)),
            pl.BlockSpec((GATE_TB, 1), lambda i: (i, 0)),
            pl.BlockSpec((GATE_TB, 1), lambda i: (i, 0)),
            pl.BlockSpec((GATE_TB, 1), lambda i: (i, 0)),
            pl.BlockSpec((GATE_TB, 1), lambda i: (i, 0)),
            pl.BlockSpec((GATE_TB, 1), lambda i: (i, 0)),
            pl.BlockSpec((1, 16), lambda i: (0, 0)),
            pl.BlockSpec((BEXP_PAD, 1), lambda i: (0, 0)),
        ],
        out_shape=[
            jax.ShapeDtypeStruct((N, E), jnp.float32),
            jax.ShapeDtypeStruct((N, K), jnp.int32),
            jax.ShapeDtypeStruct((N, 1), jnp.int32),
            jax.ShapeDtypeStruct((N, 1), jnp.int32),
            jax.ShapeDtypeStruct((N, 1), jnp.int32),
            jax.ShapeDtypeStruct((N, 1), jnp.int32),
            jax.ShapeDtypeStruct((N, 1), jnp.float32),
            jax.ShapeDtypeStruct((N, 1), jnp.float32),
            jax.ShapeDtypeStruct((1, 16), jnp.int32),
            jax.ShapeDtypeStruct((BEXP_PAD, 1), jnp.int32),
        ],
        scratch_shapes=[pltpu.VMEM((1, E), jnp.int32)],
        compiler_params=pltpu.CompilerParams(
            dimension_semantics=("arbitrary",)),
    )(x, W_gate, bg2)

    features = pl.pallas_call(
        _backbone_body,
        grid=(N // FEAT_TB,),
        in_specs=[
            pl.BlockSpec((FEAT_TB, D), lambda i: (i, 0)),
            pl.BlockSpec((D, H), lambda i: (0, 0)),
            pl.BlockSpec((1, H), lambda i: (0, 0)),
        ],
        out_specs=pl.BlockSpec((FEAT_TB, H), lambda i: (i, 0)),
        out_shape=jax.ShapeDtypeStruct((N, H), jnp.float32),
        compiler_params=pltpu.CompilerParams(
            dimension_semantics=("parallel",)),
    )(x, W_backbone, bb2)

    _dispatch, _combine = _sc_kernels()
    g_sorted, pos0, pos1 = _dispatch(
        features, e0.reshape(N), e1.reshape(N),
        r0.reshape(N), r1.reshape(N), cnt16.reshape(16))
    bexp = bexp2d.reshape(BEXP_PAD)
    be3 = b_experts.reshape(E, 1, C)

    y_sorted = pl.pallas_call(
        _expert_body,
        grid_spec=pltpu.PrefetchScalarGridSpec(
            num_scalar_prefetch=1,
            grid=(NBLK,),
            in_specs=[
                pl.BlockSpec((GB, H), lambda i, bexp: (i, 0)),
                pl.BlockSpec((E, H, C), lambda i, bexp: (0, 0, 0)),
                pl.BlockSpec((E, 1, C), lambda i, bexp: (0, 0, 0)),
            ],
            out_specs=pl.BlockSpec((GB, CP), lambda i, bexp: (i, 0)),
        ),
        out_shape=jax.ShapeDtypeStruct((GROWS, CP), jnp.float32),
        compiler_params=pltpu.CompilerParams(
            dimension_semantics=("arbitrary",)),
    )(bexp, g_sorted, W_experts.astype(jnp.bfloat16), be3)

    outp = _combine(y_sorted, pos0, pos1, w0.reshape(N), w1.reshape(N))

    combined = pl.pallas_call(
        _slice_body,
        grid=(N // FEAT_TB,),
        in_specs=[pl.BlockSpec((FEAT_TB, CP), lambda i: (i, 0))],
        out_specs=pl.BlockSpec((FEAT_TB, C), lambda i: (i, 0)),
        out_shape=jax.ShapeDtypeStruct((N, C), jnp.float32),
        compiler_params=pltpu.CompilerParams(
            dimension_semantics=("parallel",)),
    )(outp)

    return (combined, gate_weights, top_k_indices)


# speedup vs baseline: 1.0593x; 1.0593x over previous
import functools

import jax
import jax.numpy as jnp
from jax import lax
from jax.experimental import pallas as pl
from jax.experimental.pallas import tpu as pltpu
from jax.experimental.pallas import tpu_sc as plsc

N = 8192
D = 2048
H = 2048
E = 8
K = 2
C = 1000
CP = 1024

GATE_TB = 256
FEAT_TB = 512

NC = 2
NS = 16
L = 16
NW = NC * NS
TPW = N // NW

GB = 512
NBLK = N * K // GB + E
GROWS = NBLK * GB
BEXP_PAD = 80


def _gate_body(x_ref, wg_ref, bg_ref,
               gw_ref, idx_ref, e0_ref, e1_ref, r0_ref, r1_ref,
               w0_ref, w1_ref, cnt_ref, bexp_ref, cnt_scr):
    i = pl.program_id(0)

    @pl.when(i == 0)
    def _():
        cnt_scr[...] = jnp.zeros_like(cnt_scr)

    x = x_ref[...]
    logits = jnp.dot(x, wg_ref[...], preferred_element_type=jnp.float32)
    logits = logits + bg_ref[...]
    m = jnp.max(logits, axis=1, keepdims=True)
    ex = jnp.exp(logits - m)
    probs = ex / jnp.sum(ex, axis=1, keepdims=True)
    iota_e = lax.broadcasted_iota(jnp.int32, probs.shape, 1)
    m1 = jnp.max(probs, axis=1, keepdims=True)
    a1 = jnp.min(jnp.where(probs == m1, iota_e, E), axis=1, keepdims=True)
    masked = jnp.where(iota_e == a1, -jnp.inf, probs)
    m2 = jnp.max(masked, axis=1, keepdims=True)
    a2 = jnp.min(jnp.where(masked == m2, iota_e, E), axis=1, keepdims=True)
    gw = jnp.where(iota_e == a1, m1, 0.0) + jnp.where(iota_e == a2, m2, 0.0)
    gw_ref[...] = gw
    idx_ref[...] = jnp.concatenate([a1, a2], axis=1)
    e0_ref[...] = a1
    e1_ref[...] = a2
    w0_ref[...] = m1
    w1_ref[...] = m2

    sel = (jnp.where(iota_e == a1, 1.0, 0.0)
           + jnp.where(iota_e == a2, 1.0, 0.0))
    row_i = lax.broadcasted_iota(jnp.int32, (GATE_TB, GATE_TB), 0)
    col_i = lax.broadcasted_iota(jnp.int32, (GATE_TB, GATE_TB), 1)
    lt = jnp.where(col_i < row_i, 1.0, 0.0)
    rank_prev = jnp.dot(lt, sel, preferred_element_type=jnp.float32)
    runcnt = cnt_scr[...]
    rpt = rank_prev.astype(jnp.int32) + runcnt
    r0_ref[...] = jnp.sum(jnp.where(iota_e == a1, rpt, 0),
                          axis=1, keepdims=True)
    r1_ref[...] = jnp.sum(jnp.where(iota_e == a2, rpt, 0),
                          axis=1, keepdims=True)
    newcnt = runcnt + jnp.sum(sel, axis=0, keepdims=True).astype(jnp.int32)
    cnt_scr[...] = newcnt
    cnt_ref[...] = jnp.concatenate(
        [newcnt, jnp.zeros((1, 16 - E), jnp.int32)], axis=1)

    cntf = newcnt.astype(jnp.float32)
    pb = jnp.floor((cntf + (GB - 1)) * (1.0 / GB))
    r8 = lax.broadcasted_iota(jnp.int32, (E, E), 0)
    c8 = lax.broadcasted_iota(jnp.int32, (E, E), 1)
    slt = jnp.where(r8 < c8, 1.0, 0.0)
    excl_b = jnp.dot(pb, slt, preferred_element_type=jnp.float32)
    iv = lax.broadcasted_iota(jnp.int32, (BEXP_PAD, E), 0).astype(jnp.float32)
    hits = jnp.where(iv >= excl_b, 1, 0)
    bexp_ref[...] = jnp.sum(hits, axis=1, keepdims=True).astype(jnp.int32) - 1


def _backbone_body(x_ref, wb_ref, bb_ref, feat_ref):
    acc = jnp.dot(x_ref[...], wb_ref[...], preferred_element_type=jnp.float32)
    feat_ref[...] = jnp.maximum(acc + bb_ref[...], 0.0)


def _dispatch_body(feat_hbm, e0_hbm, e1_hbm, r0_hbm, r1_hbm, cnt_hbm,
                   g_hbm, pos0_hbm, pos1_hbm,
                   cntv, exclv, e0v, e1v, r0v, r1v, p0v, p1v,
                   fbuf0, fbuf1, lsem0, lsem1, ssem0, ssem1):
    wid = lax.axis_index("s") * NC + lax.axis_index("c")
    base = wid * TPW
    nchunk = TPW // L
    bufs = (fbuf0, fbuf1)
    lsems = (lsem0, lsem1)
    ssems = (ssem0, ssem1)

    pltpu.sync_copy(cnt_hbm, cntv)
    cnt = cntv[...]
    padded = ((cnt + (GB - 1)) // GB) * GB
    incl = plsc.cumsum(padded)
    exclv[...] = incl - padded

    pltpu.sync_copy(e0_hbm.at[pl.ds(base, TPW)], e0v)
    pltpu.sync_copy(e1_hbm.at[pl.ds(base, TPW)], e1v)
    pltpu.sync_copy(r0_hbm.at[pl.ds(base, TPW)], r0v)
    pltpu.sync_copy(r1_hbm.at[pl.ds(base, TPW)], r1v)

    for t in range(nchunk):
        sl = pl.ds(t * L, L)
        p0v[sl] = plsc.load_gather(exclv, [e0v[sl]]) + r0v[sl]
        p1v[sl] = plsc.load_gather(exclv, [e1v[sl]]) + r1v[sl]

    pltpu.sync_copy(p0v, pos0_hbm.at[pl.ds(base, TPW)])
    pltpu.sync_copy(p1v, pos1_hbm.at[pl.ds(base, TPW)])

    def _load(t):
        pltpu.make_async_copy(
            feat_hbm.at[pl.ds(base + t * L, L), :],
            bufs[t % 2], lsems[t % 2]).start()

    def _wait_load(t):
        pltpu.make_async_copy(
            feat_hbm.at[pl.ds(base + t * L, L), :],
            bufs[t % 2], lsems[t % 2]).wait()

    def _wait_scatter(t):
        sl = pl.ds(t * L, L)
        pltpu.make_async_copy(bufs[t % 2], g_hbm.at[p0v[sl]],
                              ssems[t % 2]).wait()
        pltpu.make_async_copy(bufs[t % 2], g_hbm.at[p1v[sl]],
                              ssems[t % 2]).wait()

    _load(0)
    for t in range(nchunk):
        sl = pl.ds(t * L, L)
        _wait_load(t)
        pltpu.make_async_copy(bufs[t % 2], g_hbm.at[p0v[sl]],
                              ssems[t % 2]).start()
        pltpu.make_async_copy(bufs[t % 2], g_hbm.at[p1v[sl]],
                              ssems[t % 2]).start()
        if t + 1 < nchunk:
            if t >= 1:
                _wait_scatter(t - 1)
            _load(t + 1)
    _wait_scatter(nchunk - 2)
    _wait_scatter(nchunk - 1)


def _expert_body(bexp_ref, g_ref, w_ref, b_ref, y_ref):
    i = pl.program_id(0)
    e = bexp_ref[i]
    gb = g_ref[...].astype(jnp.bfloat16)
    w = w_ref[pl.ds(e, 1)][0]
    b = b_ref[pl.ds(e, 1)][0]
    y = jnp.dot(gb, w, preferred_element_type=jnp.float32)
    y_ref[...] = jnp.pad(y + b, ((0, 0), (0, CP - C)))


def _slice_body(x_ref, o_ref):
    o_ref[...] = x_ref[:, :C]


def _combine_body(y_hbm, pos0_hbm, pos1_hbm, w0_hbm, w1_hbm, out_hbm,
                  p0v, p1v, w0v, w1v, ya0, ya1, yb0, yb1, oa, ob,
                  gsem0, gsem1, osem0, osem1):
    wid = lax.axis_index("s") * NC + lax.axis_index("c")
    base = wid * TPW
    nchunk = TPW // L
    y0s = (ya0, yb0)
    y1s = (ya1, yb1)
    obufs = (oa, ob)
    gsems = (gsem0, gsem1)
    osems = (osem0, osem1)

    pltpu.sync_copy(pos0_hbm.at[pl.ds(base, TPW)], p0v)
    pltpu.sync_copy(pos1_hbm.at[pl.ds(base, TPW)], p1v)
    pltpu.sync_copy(w0_hbm.at[pl.ds(base, TPW)], w0v)
    pltpu.sync_copy(w1_hbm.at[pl.ds(base, TPW)], w1v)

    def _gather(cc):
        b = cc % 2
        sl = pl.ds(cc * L, L)
        pltpu.make_async_copy(y_hbm.at[p0v[sl]], y0s[b], gsems[b]).start()
        pltpu.make_async_copy(y_hbm.at[p1v[sl]], y1s[b], gsems[b]).start()

    def _wait_gather(cc):
        b = cc % 2
        sl = pl.ds(cc * L, L)
        pltpu.make_async_copy(y_hbm.at[p0v[sl]], y0s[b], gsems[b]).wait()
        pltpu.make_async_copy(y_hbm.at[p1v[sl]], y1s[b], gsems[b]).wait()

    def _store(cc):
        b = cc % 2
        pltpu.make_async_copy(obufs[b],
                              out_hbm.at[pl.ds(base + cc * L, L), :],
                              osems[b]).start()

    def _wait_store(cc):
        b = cc % 2
        pltpu.make_async_copy(obufs[b],
                              out_hbm.at[pl.ds(base + cc * L, L), :],
                              osems[b]).wait()

    for cc in range(nchunk):
        if cc == 0:
            _gather(0)
        _wait_gather(cc)
        if cc + 1 < nchunk:
            _gather(cc + 1)
        if cc >= 2:
            _wait_store(cc - 2)
        b = cc % 2
        ybuf0, ybuf1, obuf = y0s[b], y1s[b], obufs[b]

        def tok_body(tt, carry2, cc=cc, ybuf0=ybuf0, ybuf1=ybuf1, obuf=obuf):
            g = cc * L + tt
            w0s = plsc.load_gather(w0v, [jnp.full((L,), g, jnp.int32)])
            w1s = plsc.load_gather(w1v, [jnp.full((L,), g, jnp.int32)])
            for j in range(CP // L):
                js = pl.ds(j * L, L)
                obuf[tt, js] = w0s * ybuf0[tt, js] + w1s * ybuf1[tt, js]
            return carry2

        lax.fori_loop(0, L, tok_body, 0)
        _store(cc)
    _wait_store(nchunk - 2)
    _wait_store(nchunk - 1)


@functools.lru_cache(maxsize=1)
def _sc_kernels():
    mesh = plsc.VectorSubcoreMesh(
        core_axis_name="c", subcore_axis_name="s",
        num_cores=NC, num_subcores=NS)

    dispatch = functools.partial(
        pl.kernel,
        out_type=[
            jax.ShapeDtypeStruct((GROWS, H), jnp.float32),
            jax.ShapeDtypeStruct((N,), jnp.int32),
            jax.ShapeDtypeStruct((N,), jnp.int32),
        ],
        mesh=mesh,
        compiler_params=pltpu.CompilerParams(needs_layout_passes=False),
        scratch_types=[
            pltpu.VMEM((16,), jnp.int32),
            pltpu.VMEM((16,), jnp.int32),
            pltpu.VMEM((TPW,), jnp.int32),
            pltpu.VMEM((TPW,), jnp.int32),
            pltpu.VMEM((TPW,), jnp.int32),
            pltpu.VMEM((TPW,), jnp.int32),
            pltpu.VMEM((TPW,), jnp.int32),
            pltpu.VMEM((TPW,), jnp.int32),
            pltpu.VMEM((L, H), jnp.float32),
            pltpu.VMEM((L, H), jnp.float32),
            pltpu.SemaphoreType.DMA,
            pltpu.SemaphoreType.DMA,
            pltpu.SemaphoreType.DMA,
            pltpu.SemaphoreType.DMA,
        ],
    )(_dispatch_body)

    combine = functools.partial(
        pl.kernel,
        out_type=jax.ShapeDtypeStruct((N, CP), jnp.float32),
        mesh=mesh,
        compiler_params=pltpu.CompilerParams(needs_layout_passes=False),
        scratch_types=[
            pltpu.VMEM((TPW,), jnp.int32),
            pltpu.VMEM((TPW,), jnp.int32),
            pltpu.VMEM((TPW,), jnp.float32),
            pltpu.VMEM((TPW,), jnp.float32),
            pltpu.VMEM((L, CP), jnp.float32),
            pltpu.VMEM((L, CP), jnp.float32),
            pltpu.VMEM((L, CP), jnp.float32),
            pltpu.VMEM((L, CP), jnp.float32),
            pltpu.VMEM((L, CP), jnp.float32),
            pltpu.VMEM((L, CP), jnp.float32),
            pltpu.SemaphoreType.DMA,
            pltpu.SemaphoreType.DMA,
            pltpu.SemaphoreType.DMA,
            pltpu.SemaphoreType.DMA,
        ],
    )(_combine_body)
    return dispatch, combine


def kernel(x, W_backbone, b_backbone, W_gate, b_gate, W_experts, b_experts):
    bg2 = b_gate.reshape(1, E)
    bb2 = b_backbone.reshape(1, H)

    (gate_weights, top_k_indices, e0, e1, r0, r1, w0, w1, cnt16,
     bexp2d) = pl.pallas_call(
        _gate_body,
        grid=(N // GATE_TB,),
        in_specs=[
            pl.BlockSpec((GATE_TB, D), lambda i: (i, 0)),
            pl.BlockSpec((D, E), lambda i: (0, 0)),
            pl.BlockSpec((1, E), lambda i: (0, 0)),
        ],
        out_specs=[
            pl.BlockSpec((GATE_TB, E), lambda i: (i, 0)),
            pl.BlockSpec((GATE_TB, K), lambda i: (i, 0)),
            pl.BlockSpec((GATE_TB, 1), lambda i: (i, 0)),
            pl.BlockSpec((GATE_TB, 1), lambda i: (i, 0)),
            pl.BlockSpec((GATE_TB, 1), lambda i: (i, 0)),
            pl.BlockSpec((GATE_TB, 1), lambda i: (i, 0)),
            pl.BlockSpec((GATE_TB, 1), lambda i: (i, 0)),
            pl.BlockSpec((GATE_TB, 1), lambda i: (i, 0)),
            pl.BlockSpec((1, 16), lambda i: (0, 0)),
            pl.BlockSpec((BEXP_PAD, 1), lambda i: (0, 0)),
        ],
        out_shape=[
            jax.ShapeDtypeStruct((N, E), jnp.float32),
            jax.ShapeDtypeStruct((N, K), jnp.int32),
            jax.ShapeDtypeStruct((N, 1), jnp.int32),
            jax.ShapeDtypeStruct((N, 1), jnp.int32),
            jax.ShapeDtypeStruct((N, 1), jnp.int32),
            jax.ShapeDtypeStruct((N, 1), jnp.int32),
            jax.ShapeDtypeStruct((N, 1), jnp.float32),
            jax.ShapeDtypeStruct((N, 1), jnp.float32),
            jax.ShapeDtypeStruct((1, 16), jnp.int32),
            jax.ShapeDtypeStruct((BEXP_PAD, 1), jnp.int32),
        ],
        scratch_shapes=[pltpu.VMEM((1, E), jnp.int32)],
        compiler_params=pltpu.CompilerParams(
            dimension_semantics=("arbitrary",)),
    )(x, W_gate, bg2)

    features = pl.pallas_call(
        _backbone_body,
        grid=(N // FEAT_TB,),
        in_specs=[
            pl.BlockSpec((FEAT_TB, D), lambda i: (i, 0)),
            pl.BlockSpec((D, H), lambda i: (0, 0)),
            pl.BlockSpec((1, H), lambda i: (0, 0)),
        ],
        out_specs=pl.BlockSpec((FEAT_TB, H), lambda i: (i, 0)),
        out_shape=jax.ShapeDtypeStruct((N, H), jnp.float32),
        compiler_params=pltpu.CompilerParams(
            dimension_semantics=("parallel",)),
    )(x, W_backbone, bb2)

    _dispatch, _combine = _sc_kernels()
    g_sorted, pos0, pos1 = _dispatch(
        features, e0.reshape(N), e1.reshape(N),
        r0.reshape(N), r1.reshape(N), cnt16.reshape(16))
    bexp = bexp2d.reshape(BEXP_PAD)
    be3 = b_experts.reshape(E, 1, C)

    y_sorted = pl.pallas_call(
        _expert_body,
        grid_spec=pltpu.PrefetchScalarGridSpec(
            num_scalar_prefetch=1,
            grid=(NBLK,),
            in_specs=[
                pl.BlockSpec((GB, H), lambda i, bexp: (i, 0)),
                pl.BlockSpec((E, H, C), lambda i, bexp: (0, 0, 0)),
                pl.BlockSpec((E, 1, C), lambda i, bexp: (0, 0, 0)),
            ],
            out_specs=pl.BlockSpec((GB, CP), lambda i, bexp: (i, 0)),
        ),
        out_shape=jax.ShapeDtypeStruct((GROWS, CP), jnp.float32),
        compiler_params=pltpu.CompilerParams(
            dimension_semantics=("arbitrary",)),
    )(bexp, g_sorted, W_experts.astype(jnp.bfloat16), be3)

    outp = _combine(y_sorted, pos0, pos1, w0.reshape(N), w1.reshape(N))

    combined = pl.pallas_call(
        _slice_body,
        grid=(N // FEAT_TB,),
        in_specs=[pl.BlockSpec((FEAT_TB, CP), lambda i: (i, 0))],
        out_specs=pl.BlockSpec((FEAT_TB, C), lambda i: (i, 0)),
        out_shape=jax.ShapeDtypeStruct((N, C), jnp.float32),
        compiler_params=pltpu.CompilerParams(
            dimension_semantics=("parallel",)),
    )(outp)

    return (combined, gate_weights, top_k_indices)
